# baseline (speedup 1.0000x reference)
import functools

import jax
import jax.numpy as jnp
from jax import lax
from jax.experimental import pallas as pl
from jax.experimental.pallas import tpu as pltpu
from jax.experimental.pallas import tpu_sc as plsc

N_ROIS = 32768
N_CLS = 80
NUM_WORKERS = 32
ROWS_PER_W = N_ROIS // NUM_WORKERS
IDX_MINOR = 128
N_GATHER_CHUNKS = ROWS_PER_W // IDX_MINOR

CE_BLOCK = 2048
CE_GRID = N_ROIS // CE_BLOCK


def _ce_body(x_ref, lab_ref, out_ref):
    i = pl.program_id(0)
    x = x_ref[...]
    lab = lab_ref[...]
    m = jnp.max(x, axis=-1, keepdims=True)
    s = jnp.sum(jnp.exp(x - m), axis=-1, keepdims=True)
    lse = m + jnp.log(s)
    cols = lax.broadcasted_iota(jnp.int32, x.shape, 1)
    sel = jnp.sum(jnp.where(cols == lab[:, None], x, 0.0), axis=-1,
                  keepdims=True)
    part = jnp.sum(lse - sel).reshape(1, 1)

    @pl.when(i == 0)
    def _():
        out_ref[...] = jnp.zeros((1, 1), jnp.float32)

    out_ref[...] += part


_ce_call = pl.pallas_call(
    _ce_body,
    grid=(CE_GRID,),
    in_specs=[
        pl.BlockSpec((CE_BLOCK, N_CLS + 1), lambda i: (i, 0)),
        pl.BlockSpec((CE_BLOCK,), lambda i: (i,)),
    ],
    out_specs=pl.BlockSpec((1, 1), lambda i: (0, 0)),
    out_shape=jax.ShapeDtypeStruct((1, 1), jnp.float32),
)


def _reg_body(table_hbm, lab_hbm, tgt_hbm, out_hbm,
              lab_v, idx_v, comp_v, tgt_v, acc_v, sem):
    wid = lax.axis_index("s") * 2 + lax.axis_index("c")
    base = wid * ROWS_PER_W

    pltpu.sync_copy(lab_hbm.at[pl.ds(base, ROWS_PER_W)], lab_v)
    pltpu.sync_copy(tgt_hbm.at[pl.ds(base * 4, ROWS_PER_W * 4)], tgt_v)

    iota = lax.iota(jnp.int32, 16)
    lanerow = lax.shift_right_logical(iota, 2)
    lanecol = jnp.bitwise_and(iota, 3)

    gdn = lax.GatherDimensionNumbers(
        offset_dims=(), collapsed_slice_dims=(0,), start_index_map=(0,))

    def expand4(v, sub):
        return lax.gather(v, (sub * 4 + lanerow)[:, None], gdn,
                          slice_sizes=(1,),
                          mode=lax.GatherScatterMode.PROMISE_IN_BOUNDS)

    for k in range(4 * N_GATHER_CHUNKS):
        def build(t2, carry, k=k):
            labm = lab_v[pl.ds(k * 32 + t2 * 16, 16)]
            labm = jnp.minimum(jnp.maximum(labm, 0), N_CLS - 1)
            for sub in range(4):
                rloc = k * 32 + t2 * 16 + sub * 4 + lanerow
                labx = expand4(labm, sub)
                fidx = (base + rloc) * (N_CLS * 4) + labx * 4 + lanecol
                idx_v[k, pl.ds(t2 * 64 + sub * 16, 16)] = fidx
            return carry

        lax.fori_loop(0, 2, build, 0)

    handles = []
    for k in range(4 * N_GATHER_CHUNKS):
        handles.append(pltpu.async_copy(
            table_hbm.at[idx_v.at[k]],
            comp_v.at[pl.ds(k * IDX_MINOR, IDX_MINOR)],
            sem))
    for h in handles:
        h.wait()

    def accum(t, acc):
        labm = lab_v[pl.ds(t * 16, 16)]
        m16 = jnp.where(labm < N_CLS, 1.0, 0.0).astype(jnp.float32)
        for sub in range(4):
            w = expand4(m16, sub)
            q = t * 64 + sub * 16
            d = jnp.abs(comp_v[pl.ds(q, 16)] - tgt_v[pl.ds(q, 16)])
            acc = acc + d * w
        return acc

    acc = lax.fori_loop(0, ROWS_PER_W // 16, accum,
                        jnp.zeros((16,), jnp.float32))
    acc_v[...] = acc
    pltpu.sync_copy(acc_v, out_hbm.at[wid])


@functools.lru_cache(maxsize=1)
def _reg_call():
    return functools.partial(
        pl.kernel,
        out_type=jax.ShapeDtypeStruct((NUM_WORKERS, 16), jnp.float32),
        mesh=plsc.VectorSubcoreMesh(core_axis_name="c", subcore_axis_name="s"),
        scratch_types=[
            pltpu.VMEM((ROWS_PER_W,), jnp.int32),
            pltpu.VMEM((4 * N_GATHER_CHUNKS, IDX_MINOR), jnp.int32),
            pltpu.VMEM((ROWS_PER_W * 4,), jnp.float32),
            pltpu.VMEM((ROWS_PER_W * 4,), jnp.float32),
            pltpu.VMEM((16,), jnp.float32),
            pltpu.SemaphoreType.DMA,
        ],
    )(_reg_body)


def kernel(cls_preds, reg_preds, cls_labels, reg_targets):
    labels = cls_labels.astype(jnp.int32)
    table = reg_preds.reshape(N_ROIS * N_CLS * 4)
    tgt_flat = reg_targets.reshape(N_ROIS * 4)

    reg_parts = _reg_call()(table, labels, tgt_flat)
    cls_sum = _ce_call(cls_preds, labels)

    cls_loss = cls_sum[0, 0] / N_ROIS
    reg_loss = jnp.sum(reg_parts) / N_ROIS
    return cls_loss, reg_loss


# speedup vs baseline: 1.4768x; 1.4768x over previous
import functools

import jax
import jax.numpy as jnp
from jax import lax
from jax.experimental import pallas as pl
from jax.experimental.pallas import tpu as pltpu
from jax.experimental.pallas import tpu_sc as plsc

N_ROIS = 32768
N_CLS = 80
NUM_WORKERS = 32
ROWS_PER_W = N_ROIS // NUM_WORKERS
SLAB = 128
N_SLABS = ROWS_PER_W // SLAB

CE_BLOCK = 2048
CE_GRID = N_ROIS // CE_BLOCK


def _ce_body(x_ref, lab_ref, out_ref):
    i = pl.program_id(0)
    x = x_ref[...]
    lab = lab_ref[...]
    m = jnp.max(x, axis=-1, keepdims=True)
    s = jnp.sum(jnp.exp(x - m), axis=-1, keepdims=True)
    cols = lax.broadcasted_iota(jnp.int32, x.shape, 1)
    sel = jnp.where(cols == lab, x, 0.0)
    part = (jnp.sum(m) + jnp.sum(jnp.log(s)) - jnp.sum(sel)).reshape(1, 1)

    @pl.when(i == 0)
    def _():
        out_ref[...] = jnp.zeros((1, 1), jnp.float32)

    out_ref[...] += part


_ce_call = pl.pallas_call(
    _ce_body,
    grid=(CE_GRID,),
    in_specs=[
        pl.BlockSpec((CE_BLOCK, N_CLS + 1), lambda i: (i, 0)),
        pl.BlockSpec((CE_BLOCK, 1), lambda i: (i, 0)),
    ],
    out_specs=pl.BlockSpec((1, 1), lambda i: (0, 0)),
    out_shape=jax.ShapeDtypeStruct((1, 1), jnp.float32),
)


def _reg_body(tab_hbm, lab_hbm, tgt_hbm, out_hbm,
              lab_v, slab_v, tgt_v, acc_v, sem0, sem1):
    wid = lax.axis_index("s") * 2 + lax.axis_index("c")
    base = wid * ROWS_PER_W

    pltpu.sync_copy(lab_hbm.at[pl.ds(base, ROWS_PER_W)], lab_v)
    for c in range(4):
        pltpu.sync_copy(
            tgt_hbm.at[pl.ds(c * N_ROIS + base, ROWS_PER_W)],
            tgt_v.at[pl.ds(c * ROWS_PER_W, ROWS_PER_W)])

    sems = [sem0, sem1]

    def start(sl):
        return pltpu.async_copy(
            tab_hbm.at[pl.ds(base + sl * SLAB, SLAB), :],
            slab_v.at[sl % 2], sems[sl % 2])

    iota = lax.iota(jnp.int32, 16)
    lanecol = jnp.bitwise_and(iota, 3)
    gdn = lax.GatherDimensionNumbers(
        offset_dims=(), collapsed_slice_dims=(0,), start_index_map=(0,))

    handles = {0: start(0)}
    acc = jnp.float32(0.0)
    for sl in range(N_SLABS):
        if sl + 1 < N_SLABS:
            handles[sl + 1] = start(sl + 1)
        handles[sl].wait()
        slot = sl % 2

        def grp(rg, a, sl=sl, slot=slot):
            gbase = pl.multiple_of(sl * SLAB + rg * 16, 16)
            lab16 = lab_v[pl.ds(gbase, 16)]
            t16 = [tgt_v[pl.ds(c * ROWS_PER_W + gbase, 16)]
                   for c in range(4)]
            for i in range(16):
                labi = lab16[i]
                cb = jnp.minimum(jnp.maximum(labi, 0), N_CLS - 1) * 4
                cbl = pl.multiple_of(jnp.bitwise_and(cb, -16), 16)
                v = slab_v[slot, rg * 16 + i, pl.ds(cbl, 16)]
                g = lax.gather(v, (cb - cbl + lanecol)[:, None], gdn,
                               slice_sizes=(1,),
                               mode=lax.GatherScatterMode.PROMISE_IN_BOUNDS)
                s = (jnp.abs(g[0] - t16[0][i]) + jnp.abs(g[1] - t16[1][i])
                     + jnp.abs(g[2] - t16[2][i]) + jnp.abs(g[3] - t16[3][i]))
                a = a + jnp.where(labi < N_CLS, s, jnp.float32(0.0))
            return a

        acc = lax.fori_loop(0, SLAB // 16, grp, acc)

    acc_v[...] = jnp.where(iota < 1, jnp.broadcast_to(acc, (16,)), 0.0)
    pltpu.sync_copy(acc_v, out_hbm.at[wid])


@functools.lru_cache(maxsize=1)
def _reg_call():
    return functools.partial(
        pl.kernel,
        out_type=jax.ShapeDtypeStruct((NUM_WORKERS, 16), jnp.float32),
        mesh=plsc.VectorSubcoreMesh(core_axis_name="c", subcore_axis_name="s"),
        scratch_types=[
            pltpu.VMEM((ROWS_PER_W,), jnp.int32),
            pltpu.VMEM((2, SLAB, N_CLS * 4), jnp.float32),
            pltpu.VMEM((ROWS_PER_W * 4,), jnp.float32),
            pltpu.VMEM((16,), jnp.float32),
            pltpu.SemaphoreType.DMA,
            pltpu.SemaphoreType.DMA,
        ],
    )(_reg_body)


def kernel(cls_preds, reg_preds, cls_labels, reg_targets):
    labels = cls_labels.astype(jnp.int32)
    tgt_cm = reg_targets.T.reshape(N_ROIS * 4)

    reg_parts = _reg_call()(reg_preds, labels, tgt_cm)
    cls_sum = _ce_call(cls_preds, labels.reshape(N_ROIS, 1))

    cls_loss = cls_sum[0, 0] / N_ROIS
    reg_loss = jnp.sum(reg_parts) / N_ROIS
    return cls_loss, reg_loss
